# Initial kernel scaffold; baseline (speedup 1.0000x reference)
import jax
import jax.numpy as jnp
from jax.experimental import pallas as pl


def kernel(x, edge_index, W_loop, W_neigh, b):
    raise NotImplementedError("write your pallas kernel here")


# speedup vs baseline: 6.8154x; 6.8154x over previous
import functools

import jax
import jax.numpy as jnp
from jax import lax
from jax.experimental import pallas as pl
from jax.experimental.pallas import tpu as pltpu
from jax.experimental.pallas import tpu_sc as plsc


def _sc_agg_kernel(N_pad, D, NC, NS, C, G, NG):
    RPT = N_pad // NS
    ZR = 64
    mesh = plsc.VectorSubcoreMesh(core_axis_name="c", subcore_axis_name="s")

    @functools.partial(
        pl.kernel,
        mesh=mesh,
        out_type=jax.ShapeDtypeStruct((NC, NS, RPT, D), jnp.float32),
        scratch_types=[
            pltpu.VMEM((G, C), jnp.int32),
            pltpu.VMEM((G, C), jnp.int32),
            pltpu.VMEM((C, D), jnp.float32),
            pltpu.VMEM((ZR, D), jnp.float32),
            pltpu.VMEM_SHARED((N_pad, D), jnp.float32),
            pltpu.SemaphoreType.DMA,
        ],
    )
    def sc_kernel(x_hbm, src_hbm, dst_hbm, out_hbm, src_v, dst_v, rows_v,
                  zero_v, agg_sh, sem):
        c = lax.axis_index("c")
        s = lax.axis_index("s")
        wid = c * NS + s

        def zfill(i, _):
            zero_v[i // (D // 16), pl.ds((i % (D // 16)) * 16, 16)] = (
                jnp.zeros((16,), jnp.float32))
            return 0
        lax.fori_loop(0, ZR * (D // 16), zfill, 0)
        r0 = s * RPT
        for k in range(RPT // ZR):
            pltpu.sync_copy(zero_v, agg_sh.at[pl.ds(r0 + k * ZR, ZR)])
        plsc.subcore_barrier()

        def group(g, _):
            pltpu.sync_copy(src_hbm.at[wid, g], src_v)
            pltpu.sync_copy(dst_hbm.at[wid, g], dst_v)
            for j in range(G):
                pltpu.async_copy(x_hbm.at[src_v.at[j]], rows_v, sem).wait()
                pltpu.sync_copy(rows_v, agg_sh.at[dst_v.at[j]], add=True)
            return 0
        lax.fori_loop(0, NG, group, 0)
        plsc.subcore_barrier()

        pltpu.sync_copy(agg_sh.at[pl.ds(r0, RPT)], out_hbm.at[c, s])

    return sc_kernel


def _tc_finish_body(x_ref, a0_ref, a1_ref, wl_ref, wn_ref, b_ref, o_ref):
    x = x_ref[...]
    agg = a0_ref[...] + a1_ref[...]
    h = (jnp.dot(x, wl_ref[...], preferred_element_type=jnp.float32)
         + jnp.dot(agg, wn_ref[...], preferred_element_type=jnp.float32)
         + b_ref[...])
    o_ref[...] = jnp.tanh(h + x)


def kernel(x, edge_index, W_loop, W_neigh, b):
    N, D = x.shape
    E = edge_index.shape[1]

    NC, NS = 2, 16
    NW = NC * NS
    per_tile = E // NW
    C = 80
    NCH = per_tile // C
    G = 5
    NG = NCH // G
    N_pad = 10240

    src = edge_index[0].reshape(NW, NG, G, C)
    dst = edge_index[1].reshape(NW, NG, G, C)

    agg2 = _sc_agg_kernel(N_pad, D, NC, NS, C, G, NG)(x, src, dst)
    a0 = agg2[0].reshape(N_pad, D)
    a1 = agg2[1].reshape(N_pad, D)

    RB = 2000
    out = pl.pallas_call(
        _tc_finish_body,
        grid=(N // RB,),
        in_specs=[
            pl.BlockSpec((RB, D), lambda i: (i, 0)),
            pl.BlockSpec((RB, D), lambda i: (i, 0)),
            pl.BlockSpec((RB, D), lambda i: (i, 0)),
            pl.BlockSpec((D, D), lambda i: (0, 0)),
            pl.BlockSpec((D, D), lambda i: (0, 0)),
            pl.BlockSpec((1, D), lambda i: (0, 0)),
        ],
        out_specs=pl.BlockSpec((RB, D), lambda i: (i, 0)),
        out_shape=jax.ShapeDtypeStruct((N, D), jnp.float32),
    )(x, a0, a1, W_loop, W_neigh, b.reshape(1, D))
    return out
